# baseline (speedup 1.0000x reference)
import functools

import jax
import jax.numpy as jnp
from jax import lax
from jax.experimental import pallas as pl
from jax.experimental.pallas import tpu as pltpu
from jax.experimental.pallas import tpu_sc as plsc

_NUM_CORES = 2
_NUM_SUBCORES = 16
_HALF = 500000
_BLK = 2000


def _pack_body(a_ref, b_ref, o_ref):
    o_ref[:, 0:64] = a_ref[...]
    o_ref[:, 64:128] = b_ref[...]


@jax.jit
def _pack(table):
    grid = _HALF // _BLK
    return pl.pallas_call(
        _pack_body,
        grid=(grid,),
        in_specs=[
            pl.BlockSpec((_BLK, 64), lambda i: (i, 0)),
            pl.BlockSpec((_BLK, 64), lambda i: (_HALF // _BLK + i, 0)),
        ],
        out_specs=pl.BlockSpec((_BLK, 128), lambda i: (i, 0)),
        out_shape=jax.ShapeDtypeStruct((_HALF, 128), jnp.float32),
    )(table, table)


@functools.lru_cache(maxsize=None)
def _make_gather(B):
    nw = _NUM_CORES * _NUM_SUBCORES
    bpw = B // nw
    ngrp = bpw // 16
    nch = bpw // 128
    mesh = plsc.VectorSubcoreMesh(
        core_axis_name="c", subcore_axis_name="s",
        num_cores=_NUM_CORES, num_subcores=_NUM_SUBCORES)

    @functools.partial(
        pl.kernel,
        out_type=jax.ShapeDtypeStruct((B, 128), jnp.float32),
        mesh=mesh,
        scratch_types=[
            pltpu.VMEM((bpw,), jnp.int32),
            pltpu.VMEM((bpw,), jnp.int32),
            pltpu.VMEM((bpw,), jnp.int32),
            pltpu.VMEM((bpw, 128), jnp.float32),
            pltpu.SemaphoreType.DMA,
        ],
    )
    def gather(labels_hbm, packed_hbm, out_hbm,
               idx_v, gidx_v, off_v, grows_v, sem):
        wid = lax.axis_index("s") * _NUM_CORES + lax.axis_index("c")
        base = wid * bpw
        pltpu.sync_copy(labels_hbm.at[pl.ds(base, bpw)], idx_v)

        def prep(g, carry):
            vec = idx_v[pl.ds(g * 16, 16)]
            m = vec >= _HALF
            gidx_v[pl.ds(g * 16, 16)] = jnp.where(m, vec - _HALF, vec)
            off_v[pl.ds(g * 16, 16)] = jnp.where(
                m, jnp.full((16,), 64, jnp.int32),
                jnp.zeros((16,), jnp.int32))
            return carry

        lax.fori_loop(0, ngrp, prep, 0)

        copies = []
        for j in range(nch):
            c = pltpu.make_async_copy(
                packed_hbm.at[gidx_v.at[pl.ds(j * 128, 128)]],
                grows_v.at[pl.ds(j * 128, 128)],
                sem)
            c.start()
            copies.append(c)
        for c in copies:
            c.wait()

        def extract(g, carry):
            ovec = off_v[pl.ds(g * 16, 16)]
            for k in range(16):
                i = g * 16 + k
                o = ovec[k]
                for q in range(4):
                    grows_v[i, pl.ds(q * 16, 16)] = (
                        grows_v[i, pl.ds(o + q * 16, 16)])
            return carry

        lax.fori_loop(0, ngrp, extract, 0)
        pltpu.sync_copy(grows_v, out_hbm.at[pl.ds(base, bpw)])

    return gather


@jax.jit
def _embed(labels, table):
    (B,) = labels.shape
    packed = _pack(table)
    return _make_gather(B)(labels, packed)[:, :64]


def kernel(labels, train, table):
    return _embed(labels.astype(jnp.int32), table)


# speedup vs baseline: 1.8704x; 1.8704x over previous
import functools

import jax
import jax.numpy as jnp
from jax import lax
from jax.experimental import pallas as pl
from jax.experimental.pallas import tpu as pltpu
from jax.experimental.pallas import tpu_sc as plsc

_NUM_CORES = 2
_NUM_SUBCORES = 16


@functools.lru_cache(maxsize=None)
def _make_gather(B, V, D):
    nw = _NUM_CORES * _NUM_SUBCORES
    bpw = B // nw
    mesh = plsc.VectorSubcoreMesh(
        core_axis_name="c", subcore_axis_name="s",
        num_cores=_NUM_CORES, num_subcores=_NUM_SUBCORES)

    @functools.partial(
        pl.kernel,
        out_type=jax.ShapeDtypeStruct((B, D), jnp.float32),
        mesh=mesh,
        scratch_types=[
            pltpu.VMEM((bpw,), jnp.int32),
            pltpu.VMEM((bpw, D), jnp.float32),
            pltpu.SemaphoreType.DMA,
        ],
    )
    def gather(labels_hbm, table_hbm, out_hbm, idx_v, rows_v, sem):
        wid = lax.axis_index("s") * _NUM_CORES + lax.axis_index("c")
        base = wid * bpw
        pltpu.sync_copy(labels_hbm.at[pl.ds(base, bpw)], idx_v)

        def fire(g, carry):
            vec = idx_v[pl.ds(g * 16, 16)]
            for k in range(16):
                pltpu.make_async_copy(
                    table_hbm.at[pl.ds(vec[k], 1)],
                    rows_v.at[pl.ds(g * 16 + k, 1)],
                    sem).start()
            return carry

        lax.fori_loop(0, bpw // 16, fire, 0)
        pltpu.make_async_copy(
            table_hbm.at[pl.ds(0, bpw)], rows_v, sem).wait()
        pltpu.sync_copy(rows_v, out_hbm.at[pl.ds(base, bpw)])

    return gather


@jax.jit
def _embed(labels, table):
    (B,) = labels.shape
    V, D = table.shape
    return _make_gather(B, V, D)(labels, table)


def kernel(labels, train, table):
    return _embed(labels.astype(jnp.int32), table)
